# baseline (speedup 1.0000x reference)
import functools

import jax
import jax.numpy as jnp
from jax import lax
from jax.experimental import pallas as pl
from jax.experimental.pallas import tpu as pltpu
from jax.experimental.pallas import tpu_sc as plsc

N = 10000
E = 320000
D = 128
G = 64
NUM_LAYERS = 6
NP = 10240
NW = 32
NCH = 80
CH = 128
EP = NW * NCH * CH
RPT = NP // 16
RB = 2048
DEGW = 16

_mesh = plsc.VectorSubcoreMesh(core_axis_name="c", subcore_axis_name="s")


NRC = RPT // CH


def _fill_rowidx(rowidx, sid):
    base = sid * RPT
    for j in range(NRC):
        for m in range(CH // 16):
            rowidx[j, pl.ds(m * 16, 16)] = (
                base + j * CH + m * 16 + lax.iota(jnp.int32, 16))


@functools.partial(
    pl.kernel,
    out_type=jax.ShapeDtypeStruct((2, NP, DEGW), jnp.float32),
    mesh=_mesh,
    scratch_types=[
        pltpu.VMEM((CH,), jnp.int32),
        pltpu.VMEM((NRC, CH), jnp.int32),
        pltpu.VMEM((CH, DEGW), jnp.float32),
        pltpu.VMEM((CH, DEGW), jnp.float32),
        pltpu.VMEM_SHARED((NP, DEGW), jnp.float32),
    ])
def _deg_sc(c_hbm, out_hbm, cidx, rowidx, ones_v, zrows, acc):
    cid = lax.axis_index("c")
    sid = lax.axis_index("s")
    wid = sid * 2 + cid

    def fill_bufs(i, carry):
        ones_v[i, pl.ds(0, DEGW)] = jnp.ones((DEGW,), jnp.float32)
        zrows[i, pl.ds(0, DEGW)] = jnp.zeros((DEGW,), jnp.float32)
        return carry
    lax.fori_loop(0, CH, fill_bufs, 0)
    _fill_rowidx(rowidx, sid)

    for j in range(NRC):
        pltpu.sync_copy(zrows, acc.at[rowidx.at[j]])
    plsc.subcore_barrier()

    def body(k, carry):
        pltpu.sync_copy(c_hbm.at[wid, k], cidx)
        pltpu.sync_copy(ones_v, acc.at[cidx], add=True)
        return carry
    lax.fori_loop(0, NCH, body, 0)

    plsc.subcore_barrier()
    for j in range(NRC):
        pltpu.sync_copy(acc.at[rowidx.at[j]], zrows)
        pltpu.sync_copy(zrows, out_hbm.at[cid, pl.ds(sid * RPT + j * CH, CH)])


@functools.partial(
    pl.kernel,
    out_type=jax.ShapeDtypeStruct((2, NP, D), jnp.float32),
    mesh=_mesh,
    scratch_types=[
        pltpu.VMEM((2, CH), jnp.int32),
        pltpu.VMEM((2, CH), jnp.int32),
        pltpu.VMEM((NRC, CH), jnp.int32),
        pltpu.VMEM((CH, D), jnp.float32),
        pltpu.VMEM((CH, D), jnp.float32),
        pltpu.VMEM_SHARED((NP, D), jnp.float32),
        pltpu.SemaphoreType.DMA,
        pltpu.SemaphoreType.DMA,
        pltpu.SemaphoreType.DMA,
        pltpu.SemaphoreType.DMA,
    ])
def _agg_sc(s_hbm, e_hbm, out_hbm, ibuf0, ibuf1, rowidx, rows0, rows1,
            acc, sg0, sg1, si0, si1):
    cid = lax.axis_index("c")
    sid = lax.axis_index("s")
    wid = sid * 2 + cid

    def fill_zero(i, carry):
        for j in range(D // 16):
            rows0[i, pl.ds(j * 16, 16)] = jnp.zeros((16,), jnp.float32)
        return carry
    lax.fori_loop(0, CH, fill_zero, 0)
    _fill_rowidx(rowidx, sid)

    for j in range(NRC):
        pltpu.sync_copy(rows0, acc.at[rowidx.at[j]])
    plsc.subcore_barrier()

    pltpu.sync_copy(e_hbm.at[wid, 0], ibuf0)
    pltpu.async_copy(s_hbm.at[ibuf0.at[0]], rows0, sg0)

    def body(h, carry):
        k1 = 2 * h + 1
        pltpu.make_async_copy(s_hbm.at[ibuf0.at[0]], rows0, sg0).wait()
        pltpu.sync_copy(e_hbm.at[wid, k1], ibuf1)
        pltpu.async_copy(s_hbm.at[ibuf1.at[0]], rows1, sg1)
        pltpu.sync_copy(rows0, acc.at[ibuf0.at[1]], add=True)
        pltpu.sync_copy(e_hbm.at[wid, k1 + 1], ibuf0)
        pltpu.make_async_copy(s_hbm.at[ibuf1.at[0]], rows1, sg1).wait()
        pltpu.async_copy(s_hbm.at[ibuf0.at[0]], rows0, sg0)
        pltpu.sync_copy(rows1, acc.at[ibuf1.at[1]], add=True)
        return carry
    lax.fori_loop(0, NCH // 2, body, 0)
    pltpu.make_async_copy(s_hbm.at[ibuf0.at[0]], rows0, sg0).wait()

    plsc.subcore_barrier()
    for j in range(NRC):
        pltpu.sync_copy(acc.at[rowidx.at[j]], rows0)
        pltpu.sync_copy(rows0, out_hbm.at[cid, pl.ds(sid * RPT + j * CH, CH)])


def _prep_body(degp_ref, dinv_ref):
    d = degp_ref[0, :, 0:1] + degp_ref[1, :, 0:1] + 1.0
    dinv_ref[...] = lax.rsqrt(d)


def _mm0_body(x_ref, w_ref, dinv_ref, o_ref):
    o_ref[...] = dinv_ref[...] * jnp.dot(
        x_ref[...], w_ref[...], preferred_element_type=jnp.float32)


def _layer_body(p_ref, s_ref, dinv_ref, b_ref, w_ref, o_ref):
    di = dinv_ref[...]
    h = jnp.maximum(di * (p_ref[0] + p_ref[1] + s_ref[...]) + b_ref[...], 0.0)
    o_ref[...] = di * jnp.dot(h, w_ref[...], preferred_element_type=jnp.float32)


def _pool_body(p_ref, s_ref, dinv_ref, b_ref, batch_ref, sums_ref, cnt_ref):
    @pl.when(pl.program_id(0) == 0)
    def _():
        sums_ref[...] = jnp.zeros_like(sums_ref)
        cnt_ref[...] = jnp.zeros_like(cnt_ref)
    di = dinv_ref[...]
    h = jnp.maximum(di * (p_ref[0] + p_ref[1] + s_ref[...]) + b_ref[...], 0.0)
    oh = (batch_ref[...] == lax.broadcasted_iota(jnp.int32, (1, G), 1)
          ).astype(jnp.float32)
    sums_ref[...] += lax.dot_general(
        oh, h, (((0,), (0,)), ((), ())), preferred_element_type=jnp.float32)
    cnt_ref[...] += lax.dot_general(
        oh, jnp.ones((RB, 1), jnp.float32), (((0,), (0,)), ((), ())),
        preferred_element_type=jnp.float32)


def _lin_body(sums_ref, cnt_ref, w_ref, b_ref, o_ref):
    pooled = sums_ref[...] / jnp.maximum(cnt_ref[...], 1.0)
    o_ref[...] = jnp.dot(
        pooled, w_ref[...], preferred_element_type=jnp.float32) + b_ref[...]


def kernel(x, edge_index, batch, w0, w1, w2, w3, w4, w5,
           b0, b1, b2, b3, b4, b5, lin_w, lin_b):
    ws = [w0, w1, w2, w3, w4, w5]
    bs = [b0, b1, b2, b3, b4, b5]
    r = edge_index[0]
    c = edge_index[1]
    idx_pad = jnp.full((EP - E,), N, jnp.int32)
    chunk_pad = jnp.full((NW, 2, CH), N, jnp.int32)
    r3 = jnp.concatenate([r, idx_pad]).reshape(NW, NCH, CH)
    c3 = jnp.concatenate([c, idx_pad]).reshape(NW, NCH, CH)
    r3 = jnp.concatenate([r3, chunk_pad], axis=1)
    c3 = jnp.concatenate([c3, chunk_pad], axis=1)
    e3 = jnp.stack([r3, c3], axis=2)
    x_pad = jnp.pad(x, ((0, NP - N), (0, 0)))
    batch2 = jnp.pad(batch, (0, NP - N), constant_values=G).reshape(NP, 1)

    degp = _deg_sc(c3)

    dinv = pl.pallas_call(
        _prep_body,
        out_shape=jax.ShapeDtypeStruct((NP, 1), jnp.float32),
    )(degp)

    grid = (NP // RB,)
    s = pl.pallas_call(
        _mm0_body,
        grid=grid,
        in_specs=[pl.BlockSpec((RB, D), lambda i: (i, 0)),
                  pl.BlockSpec((D, D), lambda i: (0, 0)),
                  pl.BlockSpec((RB, 1), lambda i: (i, 0))],
        out_specs=pl.BlockSpec((RB, D), lambda i: (i, 0)),
        out_shape=jax.ShapeDtypeStruct((NP, D), jnp.float32),
    )(x_pad, ws[0], dinv)

    for i in range(NUM_LAYERS):
        p = _agg_sc(s, e3)
        if i < NUM_LAYERS - 1:
            s = pl.pallas_call(
                _layer_body,
                grid=grid,
                in_specs=[pl.BlockSpec((2, RB, D), lambda i: (0, i, 0)),
                          pl.BlockSpec((RB, D), lambda i: (i, 0)),
                          pl.BlockSpec((RB, 1), lambda i: (i, 0)),
                          pl.BlockSpec((1, D), lambda i: (0, 0)),
                          pl.BlockSpec((D, D), lambda i: (0, 0))],
                out_specs=pl.BlockSpec((RB, D), lambda i: (i, 0)),
                out_shape=jax.ShapeDtypeStruct((NP, D), jnp.float32),
            )(p, s, dinv, bs[i].reshape(1, D), ws[i + 1])
        else:
            sums, cnt = pl.pallas_call(
                _pool_body,
                grid=grid,
                in_specs=[pl.BlockSpec((2, RB, D), lambda i: (0, i, 0)),
                          pl.BlockSpec((RB, D), lambda i: (i, 0)),
                          pl.BlockSpec((RB, 1), lambda i: (i, 0)),
                          pl.BlockSpec((1, D), lambda i: (0, 0)),
                          pl.BlockSpec((RB, 1), lambda i: (i, 0))],
                out_specs=[pl.BlockSpec((G, D), lambda i: (0, 0)),
                           pl.BlockSpec((G, 1), lambda i: (0, 0))],
                out_shape=[jax.ShapeDtypeStruct((G, D), jnp.float32),
                           jax.ShapeDtypeStruct((G, 1), jnp.float32)],
            )(p, s, dinv, bs[i].reshape(1, D), batch2)

    out = pl.pallas_call(
        _lin_body,
        out_shape=jax.ShapeDtypeStruct((G, G), jnp.float32),
    )(sums, cnt, lin_w, lin_b.reshape(1, G))
    return out


# speedup vs baseline: 1.2108x; 1.2108x over previous
import functools

import jax
import jax.numpy as jnp
from jax import lax
from jax.experimental import pallas as pl
from jax.experimental.pallas import tpu as pltpu
from jax.experimental.pallas import tpu_sc as plsc

N = 10000
E = 320000
D = 128
G = 64
NUM_LAYERS = 6
NP = 10240
NW = 32
NCH = 80
CH = 128
EP = NW * NCH * CH
RPT = NP // 16
RB = 2048
DEGW = 16

_mesh = plsc.VectorSubcoreMesh(core_axis_name="c", subcore_axis_name="s")


NRC = RPT // CH


def _fill_rowidx(rowidx, sid):
    base = sid * RPT
    for j in range(NRC):
        for m in range(CH // 16):
            rowidx[j, pl.ds(m * 16, 16)] = (
                base + j * CH + m * 16 + lax.iota(jnp.int32, 16))


@functools.partial(
    pl.kernel,
    out_type=jax.ShapeDtypeStruct((2, NP, DEGW), jnp.float32),
    mesh=_mesh,
    scratch_types=[
        pltpu.VMEM((CH,), jnp.int32),
        pltpu.VMEM((NRC, CH), jnp.int32),
        pltpu.VMEM((CH, DEGW), jnp.float32),
        pltpu.VMEM((CH, DEGW), jnp.float32),
        pltpu.VMEM_SHARED((NP, DEGW), jnp.float32),
    ])
def _deg_sc(c_hbm, out_hbm, cidx, rowidx, ones_v, zrows, acc):
    cid = lax.axis_index("c")
    sid = lax.axis_index("s")
    wid = sid * 2 + cid

    def fill_bufs(i, carry):
        ones_v[i, pl.ds(0, DEGW)] = jnp.ones((DEGW,), jnp.float32)
        zrows[i, pl.ds(0, DEGW)] = jnp.zeros((DEGW,), jnp.float32)
        return carry
    lax.fori_loop(0, CH, fill_bufs, 0)
    _fill_rowidx(rowidx, sid)

    for j in range(NRC):
        pltpu.sync_copy(zrows, acc.at[rowidx.at[j]])
    plsc.subcore_barrier()

    def body(k, carry):
        pltpu.sync_copy(c_hbm.at[wid, k], cidx)
        pltpu.sync_copy(ones_v, acc.at[cidx], add=True)
        return carry
    lax.fori_loop(0, NCH, body, 0)

    plsc.subcore_barrier()
    for j in range(NRC):
        pltpu.sync_copy(acc.at[rowidx.at[j]], zrows)
        pltpu.sync_copy(zrows, out_hbm.at[cid, pl.ds(sid * RPT + j * CH, CH)])


@functools.partial(
    pl.kernel,
    out_type=jax.ShapeDtypeStruct((2, NP, D), jnp.float32),
    mesh=_mesh,
    scratch_types=[
        pltpu.VMEM((NCH + 2, 2, CH), jnp.int32),
        pltpu.VMEM((NRC, CH), jnp.int32),
        pltpu.VMEM((CH, D), jnp.float32),
        pltpu.VMEM_SHARED((NP, D), jnp.float32),
        pltpu.SemaphoreType.DMA,
    ])
def _agg_sc(s_hbm, e_hbm, out_hbm, ebuf, rowidx, rows, acc, sg0):
    cid = lax.axis_index("c")
    sid = lax.axis_index("s")
    wid = sid * 2 + cid

    def fill_zero(i, carry):
        for j in range(D // 16):
            rows[i, pl.ds(j * 16, 16)] = jnp.zeros((16,), jnp.float32)
        return carry
    lax.fori_loop(0, CH, fill_zero, 0)
    _fill_rowidx(rowidx, sid)

    for j in range(NRC):
        pltpu.sync_copy(rows, acc.at[rowidx.at[j]])
    plsc.subcore_barrier()

    pltpu.sync_copy(e_hbm.at[wid], ebuf)

    def body(k, carry):
        pltpu.async_copy(s_hbm.at[ebuf.at[k, 0]], rows, sg0).wait()
        pltpu.sync_copy(rows, acc.at[ebuf.at[k, 1]], add=True)
        return carry
    lax.fori_loop(0, NCH, body, 0)

    plsc.subcore_barrier()
    for j in range(NRC):
        pltpu.sync_copy(acc.at[rowidx.at[j]], rows)
        pltpu.sync_copy(rows, out_hbm.at[cid, pl.ds(sid * RPT + j * CH, CH)])


def _prep_body(degp_ref, dinv_ref):
    d = degp_ref[0, :, 0:1] + degp_ref[1, :, 0:1] + 1.0
    dinv_ref[...] = lax.rsqrt(d)


def _mm0_body(x_ref, w_ref, dinv_ref, o_ref):
    o_ref[...] = dinv_ref[...] * jnp.dot(
        x_ref[...], w_ref[...], preferred_element_type=jnp.float32)


def _layer_body(p_ref, s_ref, dinv_ref, b_ref, w_ref, o_ref):
    di = dinv_ref[...]
    h = jnp.maximum(di * (p_ref[0] + p_ref[1] + s_ref[...]) + b_ref[...], 0.0)
    o_ref[...] = di * jnp.dot(h, w_ref[...], preferred_element_type=jnp.float32)


def _pool_body(p_ref, s_ref, dinv_ref, b_ref, batch_ref, sums_ref, cnt_ref):
    @pl.when(pl.program_id(0) == 0)
    def _():
        sums_ref[...] = jnp.zeros_like(sums_ref)
        cnt_ref[...] = jnp.zeros_like(cnt_ref)
    di = dinv_ref[...]
    h = jnp.maximum(di * (p_ref[0] + p_ref[1] + s_ref[...]) + b_ref[...], 0.0)
    oh = (batch_ref[...] == lax.broadcasted_iota(jnp.int32, (1, G), 1)
          ).astype(jnp.float32)
    sums_ref[...] += lax.dot_general(
        oh, h, (((0,), (0,)), ((), ())), preferred_element_type=jnp.float32)
    cnt_ref[...] += lax.dot_general(
        oh, jnp.ones((RB, 1), jnp.float32), (((0,), (0,)), ((), ())),
        preferred_element_type=jnp.float32)


def _lin_body(sums_ref, cnt_ref, w_ref, b_ref, o_ref):
    pooled = sums_ref[...] / jnp.maximum(cnt_ref[...], 1.0)
    o_ref[...] = jnp.dot(
        pooled, w_ref[...], preferred_element_type=jnp.float32) + b_ref[...]


def kernel(x, edge_index, batch, w0, w1, w2, w3, w4, w5,
           b0, b1, b2, b3, b4, b5, lin_w, lin_b):
    ws = [w0, w1, w2, w3, w4, w5]
    bs = [b0, b1, b2, b3, b4, b5]
    r = edge_index[0]
    c = edge_index[1]
    idx_pad = jnp.full((EP - E,), N, jnp.int32)
    chunk_pad = jnp.full((NW, 2, CH), N, jnp.int32)
    r3 = jnp.concatenate([r, idx_pad]).reshape(NW, NCH, CH)
    c3 = jnp.concatenate([c, idx_pad]).reshape(NW, NCH, CH)
    r3 = jnp.concatenate([r3, chunk_pad], axis=1)
    c3 = jnp.concatenate([c3, chunk_pad], axis=1)
    e3 = jnp.stack([r3, c3], axis=2)
    x_pad = jnp.pad(x, ((0, NP - N), (0, 0)))
    batch2 = jnp.pad(batch, (0, NP - N), constant_values=G).reshape(NP, 1)

    degp = _deg_sc(c3)

    dinv = pl.pallas_call(
        _prep_body,
        out_shape=jax.ShapeDtypeStruct((NP, 1), jnp.float32),
    )(degp)

    grid = (NP // RB,)
    s = pl.pallas_call(
        _mm0_body,
        grid=grid,
        in_specs=[pl.BlockSpec((RB, D), lambda i: (i, 0)),
                  pl.BlockSpec((D, D), lambda i: (0, 0)),
                  pl.BlockSpec((RB, 1), lambda i: (i, 0))],
        out_specs=pl.BlockSpec((RB, D), lambda i: (i, 0)),
        out_shape=jax.ShapeDtypeStruct((NP, D), jnp.float32),
    )(x_pad, ws[0], dinv)

    for i in range(NUM_LAYERS):
        p = _agg_sc(s, e3)
        if i < NUM_LAYERS - 1:
            s = pl.pallas_call(
                _layer_body,
                grid=grid,
                in_specs=[pl.BlockSpec((2, RB, D), lambda i: (0, i, 0)),
                          pl.BlockSpec((RB, D), lambda i: (i, 0)),
                          pl.BlockSpec((RB, 1), lambda i: (i, 0)),
                          pl.BlockSpec((1, D), lambda i: (0, 0)),
                          pl.BlockSpec((D, D), lambda i: (0, 0))],
                out_specs=pl.BlockSpec((RB, D), lambda i: (i, 0)),
                out_shape=jax.ShapeDtypeStruct((NP, D), jnp.float32),
            )(p, s, dinv, bs[i].reshape(1, D), ws[i + 1])
        else:
            sums, cnt = pl.pallas_call(
                _pool_body,
                grid=grid,
                in_specs=[pl.BlockSpec((2, RB, D), lambda i: (0, i, 0)),
                          pl.BlockSpec((RB, D), lambda i: (i, 0)),
                          pl.BlockSpec((RB, 1), lambda i: (i, 0)),
                          pl.BlockSpec((1, D), lambda i: (0, 0)),
                          pl.BlockSpec((RB, 1), lambda i: (i, 0))],
                out_specs=[pl.BlockSpec((G, D), lambda i: (0, 0)),
                           pl.BlockSpec((G, 1), lambda i: (0, 0))],
                out_shape=[jax.ShapeDtypeStruct((G, D), jnp.float32),
                           jax.ShapeDtypeStruct((G, 1), jnp.float32)],
            )(p, s, dinv, bs[i].reshape(1, D), batch2)

    out = pl.pallas_call(
        _lin_body,
        out_shape=jax.ShapeDtypeStruct((G, G), jnp.float32),
    )(sums, cnt, lin_w, lin_b.reshape(1, G))
    return out


# speedup vs baseline: 1.3773x; 1.1375x over previous
import functools

import jax
import jax.numpy as jnp
from jax import lax
from jax.experimental import pallas as pl
from jax.experimental.pallas import tpu as pltpu
from jax.experimental.pallas import tpu_sc as plsc

N = 10000
E = 320000
D = 128
G = 64
NUM_LAYERS = 6
NP = 10240
NW = 32
NCH = 80
CH = 128
EP = NW * NCH * CH
NCH_FAST = 110
NCH_SLOW = 50
NCHM = NCH_FAST
RPT = NP // 16
RB = 2048
DEGW = 16

_mesh = plsc.VectorSubcoreMesh(core_axis_name="c", subcore_axis_name="s")


NRC = RPT // CH


def _fill_rowidx(rowidx, sid):
    base = sid * RPT
    for j in range(NRC):
        for m in range(CH // 16):
            rowidx[j, pl.ds(m * 16, 16)] = (
                base + j * CH + m * 16 + lax.iota(jnp.int32, 16))


@functools.partial(
    pl.kernel,
    out_type=jax.ShapeDtypeStruct((2, NP, DEGW), jnp.float32),
    mesh=_mesh,
    scratch_types=[
        pltpu.VMEM((CH,), jnp.int32),
        pltpu.VMEM((NRC, CH), jnp.int32),
        pltpu.VMEM((CH, DEGW), jnp.float32),
        pltpu.VMEM((CH, DEGW), jnp.float32),
        pltpu.VMEM_SHARED((NP, DEGW), jnp.float32),
    ])
def _deg_sc(c_hbm, out_hbm, cidx, rowidx, ones_v, zrows, acc):
    cid = lax.axis_index("c")
    sid = lax.axis_index("s")
    wid = sid * 2 + cid
    nch = NCH_SLOW + (1 - cid) * (NCH_FAST - NCH_SLOW)

    def fill_bufs(i, carry):
        ones_v[i, pl.ds(0, DEGW)] = jnp.ones((DEGW,), jnp.float32)
        zrows[i, pl.ds(0, DEGW)] = jnp.zeros((DEGW,), jnp.float32)
        return carry
    lax.fori_loop(0, CH, fill_bufs, 0)
    _fill_rowidx(rowidx, sid)

    for j in range(NRC):
        pltpu.sync_copy(zrows, acc.at[rowidx.at[j]])
    plsc.subcore_barrier()

    def body(k, carry):
        pltpu.sync_copy(c_hbm.at[wid, k], cidx)
        pltpu.sync_copy(ones_v, acc.at[cidx], add=True)
        return carry
    lax.cond(cid == 0,
             lambda: lax.fori_loop(0, NCH_FAST, body, 0),
             lambda: lax.fori_loop(0, NCH_SLOW, body, 0))

    plsc.subcore_barrier()
    for j in range(NRC):
        pltpu.sync_copy(acc.at[rowidx.at[j]], zrows)
        pltpu.sync_copy(zrows, out_hbm.at[cid, pl.ds(sid * RPT + j * CH, CH)])


@functools.partial(
    pl.kernel,
    out_type=jax.ShapeDtypeStruct((2, NP, D), jnp.float32),
    mesh=_mesh,
    scratch_types=[
        pltpu.VMEM((NCHM, CH), jnp.int32),
        pltpu.VMEM((CH,), jnp.int32),
        pltpu.VMEM((NRC, CH), jnp.int32),
        pltpu.VMEM((CH, D), jnp.float32),
        pltpu.VMEM_SHARED((NP, D), jnp.float32),
        pltpu.SemaphoreType.DMA,
    ])
def _agg_sc(s_hbm, r_hbm, c_hbm, out_hbm, rbuf, cidx, rowidx, rows, acc, sg0):
    cid = lax.axis_index("c")
    sid = lax.axis_index("s")
    wid = sid * 2 + cid
    nch = NCH_SLOW + (1 - cid) * (NCH_FAST - NCH_SLOW)

    def fill_zero(i, carry):
        for j in range(D // 16):
            rows[i, pl.ds(j * 16, 16)] = jnp.zeros((16,), jnp.float32)
        return carry
    lax.fori_loop(0, CH, fill_zero, 0)
    _fill_rowidx(rowidx, sid)

    for j in range(NRC):
        pltpu.sync_copy(rows, acc.at[rowidx.at[j]])
    plsc.subcore_barrier()

    pltpu.sync_copy(r_hbm.at[wid], rbuf)

    def body(k, carry):
        pltpu.sync_copy(c_hbm.at[wid, k], cidx)
        pltpu.async_copy(s_hbm.at[rbuf.at[k]], rows, sg0).wait()
        pltpu.sync_copy(rows, acc.at[cidx], add=True)
        return carry
    lax.cond(cid == 0,
             lambda: lax.fori_loop(0, NCH_FAST, body, 0),
             lambda: lax.fori_loop(0, NCH_SLOW, body, 0))

    plsc.subcore_barrier()
    for j in range(NRC):
        pltpu.sync_copy(acc.at[rowidx.at[j]], rows)
        pltpu.sync_copy(rows, out_hbm.at[cid, pl.ds(sid * RPT + j * CH, CH)])


def _prep_body(degp_ref, dinv_ref):
    d = degp_ref[0, :, 0:1] + degp_ref[1, :, 0:1] + 1.0
    dinv_ref[...] = lax.rsqrt(d)


def _mm0_body(x_ref, w_ref, dinv_ref, o_ref):
    o_ref[...] = dinv_ref[...] * jnp.dot(
        x_ref[...], w_ref[...], preferred_element_type=jnp.float32)


def _layer_body(p_ref, s_ref, dinv_ref, b_ref, w_ref, o_ref):
    di = dinv_ref[...]
    h = jnp.maximum(di * (p_ref[0] + p_ref[1] + s_ref[...]) + b_ref[...], 0.0)
    o_ref[...] = di * jnp.dot(h, w_ref[...], preferred_element_type=jnp.float32)


def _pool_body(p_ref, s_ref, dinv_ref, b_ref, batch_ref, sums_ref, cnt_ref):
    @pl.when(pl.program_id(0) == 0)
    def _():
        sums_ref[...] = jnp.zeros_like(sums_ref)
        cnt_ref[...] = jnp.zeros_like(cnt_ref)
    di = dinv_ref[...]
    h = jnp.maximum(di * (p_ref[0] + p_ref[1] + s_ref[...]) + b_ref[...], 0.0)
    oh = (batch_ref[...] == lax.broadcasted_iota(jnp.int32, (1, G), 1)
          ).astype(jnp.float32)
    sums_ref[...] += lax.dot_general(
        oh, h, (((0,), (0,)), ((), ())), preferred_element_type=jnp.float32)
    cnt_ref[...] += lax.dot_general(
        oh, jnp.ones((RB, 1), jnp.float32), (((0,), (0,)), ((), ())),
        preferred_element_type=jnp.float32)


def _lin_body(sums_ref, cnt_ref, w_ref, b_ref, o_ref):
    pooled = sums_ref[...] / jnp.maximum(cnt_ref[...], 1.0)
    o_ref[...] = jnp.dot(
        pooled, w_ref[...], preferred_element_type=jnp.float32) + b_ref[...]


def kernel(x, edge_index, batch, w0, w1, w2, w3, w4, w5,
           b0, b1, b2, b3, b4, b5, lin_w, lin_b):
    ws = [w0, w1, w2, w3, w4, w5]
    bs = [b0, b1, b2, b3, b4, b5]
    r = edge_index[0]
    c = edge_index[1]
    def _split(v):
        flat = jnp.concatenate([v, jnp.full((EP - E,), N, jnp.int32)])
        flat = flat.reshape(NW * NCH, CH)
        a = flat[:16 * NCH_FAST].reshape(16, NCH_FAST, CH)
        b = flat[16 * NCH_FAST:].reshape(16, NCH_SLOW, CH)
        b = jnp.concatenate(
            [b, jnp.full((16, NCHM - NCH_SLOW, CH), N, jnp.int32)], axis=1)
        return jnp.stack([a, b], axis=1).reshape(NW, NCHM, CH)
    r3 = _split(r)
    c3 = _split(c)
    x_pad = jnp.pad(x, ((0, NP - N), (0, 0)))
    batch2 = jnp.pad(batch, (0, NP - N), constant_values=G).reshape(NP, 1)

    degp = _deg_sc(c3)

    dinv = pl.pallas_call(
        _prep_body,
        out_shape=jax.ShapeDtypeStruct((NP, 1), jnp.float32),
    )(degp)

    grid = (NP // RB,)
    s = pl.pallas_call(
        _mm0_body,
        grid=grid,
        in_specs=[pl.BlockSpec((RB, D), lambda i: (i, 0)),
                  pl.BlockSpec((D, D), lambda i: (0, 0)),
                  pl.BlockSpec((RB, 1), lambda i: (i, 0))],
        out_specs=pl.BlockSpec((RB, D), lambda i: (i, 0)),
        out_shape=jax.ShapeDtypeStruct((NP, D), jnp.float32),
    )(x_pad, ws[0], dinv)

    for i in range(NUM_LAYERS):
        p = _agg_sc(s, r3, c3)
        if i < NUM_LAYERS - 1:
            s = pl.pallas_call(
                _layer_body,
                grid=grid,
                in_specs=[pl.BlockSpec((2, RB, D), lambda i: (0, i, 0)),
                          pl.BlockSpec((RB, D), lambda i: (i, 0)),
                          pl.BlockSpec((RB, 1), lambda i: (i, 0)),
                          pl.BlockSpec((1, D), lambda i: (0, 0)),
                          pl.BlockSpec((D, D), lambda i: (0, 0))],
                out_specs=pl.BlockSpec((RB, D), lambda i: (i, 0)),
                out_shape=jax.ShapeDtypeStruct((NP, D), jnp.float32),
            )(p, s, dinv, bs[i].reshape(1, D), ws[i + 1])
        else:
            sums, cnt = pl.pallas_call(
                _pool_body,
                grid=grid,
                in_specs=[pl.BlockSpec((2, RB, D), lambda i: (0, i, 0)),
                          pl.BlockSpec((RB, D), lambda i: (i, 0)),
                          pl.BlockSpec((RB, 1), lambda i: (i, 0)),
                          pl.BlockSpec((1, D), lambda i: (0, 0)),
                          pl.BlockSpec((RB, 1), lambda i: (i, 0))],
                out_specs=[pl.BlockSpec((G, D), lambda i: (0, 0)),
                           pl.BlockSpec((G, 1), lambda i: (0, 0))],
                out_shape=[jax.ShapeDtypeStruct((G, D), jnp.float32),
                           jax.ShapeDtypeStruct((G, 1), jnp.float32)],
            )(p, s, dinv, bs[i].reshape(1, D), batch2)

    out = pl.pallas_call(
        _lin_body,
        out_shape=jax.ShapeDtypeStruct((G, G), jnp.float32),
    )(sums, cnt, lin_w, lin_b.reshape(1, G))
    return out


# speedup vs baseline: 2.9412x; 2.1355x over previous
import functools

import jax
import jax.numpy as jnp
from jax import lax
from jax.experimental import pallas as pl
from jax.experimental.pallas import tpu as pltpu
from jax.experimental.pallas import tpu_sc as plsc

N = 10000
E = 320000
D = 128
G = 64
NUM_LAYERS = 6
NP = 10240
NW = 32
NCH = 80
CH = 128
EP = NW * NCH * CH
RPT = NP // 16
RB = 2048
DEGW = 16

_mesh = plsc.VectorSubcoreMesh(core_axis_name="c", subcore_axis_name="s")


NRC = RPT // CH


def _fill_rowidx(rowidx, sid):
    base = sid * RPT
    for j in range(NRC):
        for m in range(CH // 16):
            rowidx[j, pl.ds(m * 16, 16)] = (
                base + j * CH + m * 16 + lax.iota(jnp.int32, 16))


@functools.partial(
    pl.kernel,
    out_type=jax.ShapeDtypeStruct((2, NP, DEGW), jnp.float32),
    mesh=_mesh,
    scratch_types=[
        pltpu.VMEM((CH,), jnp.int32),
        pltpu.VMEM((NRC, CH), jnp.int32),
        pltpu.VMEM((CH, DEGW), jnp.float32),
        pltpu.VMEM((CH, DEGW), jnp.float32),
        pltpu.VMEM_SHARED((NP, DEGW), jnp.float32),
    ])
def _deg_sc(c_hbm, out_hbm, cidx, rowidx, ones_v, zrows, acc):
    cid = lax.axis_index("c")
    sid = lax.axis_index("s")
    wid = sid * 2 + cid

    def fill_bufs(i, carry):
        ones_v[i, pl.ds(0, DEGW)] = jnp.ones((DEGW,), jnp.float32)
        zrows[i, pl.ds(0, DEGW)] = jnp.zeros((DEGW,), jnp.float32)
        return carry
    lax.fori_loop(0, CH, fill_bufs, 0)
    _fill_rowidx(rowidx, sid)

    for j in range(NRC):
        pltpu.sync_copy(zrows, acc.at[rowidx.at[j]])
    plsc.subcore_barrier()

    def body(k, carry):
        pltpu.sync_copy(c_hbm.at[wid, k], cidx)
        pltpu.sync_copy(ones_v, acc.at[cidx], add=True)
        return carry
    lax.fori_loop(0, NCH, body, 0)

    plsc.subcore_barrier()
    for j in range(NRC):
        pltpu.sync_copy(acc.at[rowidx.at[j]], zrows)
        pltpu.sync_copy(zrows, out_hbm.at[cid, pl.ds(sid * RPT + j * CH, CH)])


@functools.partial(
    pl.kernel,
    out_type=jax.ShapeDtypeStruct((2, NP, D), jnp.float32),
    mesh=_mesh,
    scratch_types=[
        pltpu.VMEM((NCH, CH), jnp.int32),
        pltpu.VMEM((CH,), jnp.int32),
        pltpu.VMEM((NRC, CH), jnp.int32),
        pltpu.VMEM((CH, D), jnp.float32),
        pltpu.VMEM_SHARED((NP, D), jnp.float32),
        pltpu.SemaphoreType.DMA,
    ])
def _agg_sc(s_hbm, r_hbm, c_hbm, out_hbm, rbuf, cidx, rowidx, rows, acc, sg0):
    cid = lax.axis_index("c")
    sid = lax.axis_index("s")
    wid = sid * 2 + cid

    def fill_zero(i, carry):
        for j in range(D // 16):
            rows[i, pl.ds(j * 16, 16)] = jnp.zeros((16,), jnp.float32)
        return carry
    lax.fori_loop(0, CH, fill_zero, 0)
    _fill_rowidx(rowidx, sid)

    for j in range(NRC):
        pltpu.sync_copy(rows, acc.at[rowidx.at[j]])
    plsc.subcore_barrier()

    pltpu.sync_copy(r_hbm.at[wid], rbuf)

    def body(k, carry):
        pltpu.sync_copy(c_hbm.at[wid, k], cidx)
        pltpu.async_copy(s_hbm.at[rbuf.at[k]], rows, sg0).wait()
        pltpu.sync_copy(rows, acc.at[cidx], add=True)
        return carry
    lax.fori_loop(0, NCH, body, 0)

    plsc.subcore_barrier()
    for j in range(NRC):
        pltpu.sync_copy(acc.at[rowidx.at[j]], rows)
        pltpu.sync_copy(rows, out_hbm.at[cid, pl.ds(sid * RPT + j * CH, CH)])


def _prep_body(degp_ref, dinv_ref):
    d = degp_ref[0, :, 0:1] + degp_ref[1, :, 0:1] + 1.0
    dinv_ref[...] = lax.rsqrt(d)


def _mm0_body(x_ref, w_ref, dinv_ref, o_ref):
    o_ref[...] = dinv_ref[...] * jnp.dot(
        x_ref[...], w_ref[...], preferred_element_type=jnp.float32)


def _layer_body(p_ref, s_ref, dinv_ref, b_ref, w_ref, o_ref):
    di = dinv_ref[...]
    h = jnp.maximum(di * (p_ref[0] + p_ref[1] + s_ref[...]) + b_ref[...], 0.0)
    o_ref[...] = di * jnp.dot(h, w_ref[...], preferred_element_type=jnp.float32)


def _pool_body(p_ref, s_ref, dinv_ref, b_ref, batch_ref, sums_ref, cnt_ref):
    @pl.when(pl.program_id(0) == 0)
    def _():
        sums_ref[...] = jnp.zeros_like(sums_ref)
        cnt_ref[...] = jnp.zeros_like(cnt_ref)
    di = dinv_ref[...]
    h = jnp.maximum(di * (p_ref[0] + p_ref[1] + s_ref[...]) + b_ref[...], 0.0)
    oh = (batch_ref[...] == lax.broadcasted_iota(jnp.int32, (1, G), 1)
          ).astype(jnp.float32)
    sums_ref[...] += lax.dot_general(
        oh, h, (((0,), (0,)), ((), ())), preferred_element_type=jnp.float32)
    cnt_ref[...] += lax.dot_general(
        oh, jnp.ones((RB, 1), jnp.float32), (((0,), (0,)), ((), ())),
        preferred_element_type=jnp.float32)


def _lin_body(sums_ref, cnt_ref, w_ref, b_ref, o_ref):
    pooled = sums_ref[...] / jnp.maximum(cnt_ref[...], 1.0)
    o_ref[...] = jnp.dot(
        pooled, w_ref[...], preferred_element_type=jnp.float32) + b_ref[...]


def kernel(x, edge_index, batch, w0, w1, w2, w3, w4, w5,
           b0, b1, b2, b3, b4, b5, lin_w, lin_b):
    ws = [w0, w1, w2, w3, w4, w5]
    bs = [b0, b1, b2, b3, b4, b5]
    r = edge_index[0]
    c = edge_index[1]
    idx_pad = N + (jnp.arange(EP - E, dtype=jnp.int32) % (NP - N))
    r3 = jnp.concatenate([r, idx_pad]).reshape(NW, NCH, CH)
    c3 = jnp.concatenate([c, idx_pad]).reshape(NW, NCH, CH)
    x_pad = jnp.pad(x, ((0, NP - N), (0, 0)))
    batch2 = jnp.pad(batch, (0, NP - N), constant_values=G).reshape(NP, 1)

    degp = _deg_sc(c3)

    dinv = pl.pallas_call(
        _prep_body,
        out_shape=jax.ShapeDtypeStruct((NP, 1), jnp.float32),
    )(degp)

    grid = (NP // RB,)
    s = pl.pallas_call(
        _mm0_body,
        grid=grid,
        in_specs=[pl.BlockSpec((RB, D), lambda i: (i, 0)),
                  pl.BlockSpec((D, D), lambda i: (0, 0)),
                  pl.BlockSpec((RB, 1), lambda i: (i, 0))],
        out_specs=pl.BlockSpec((RB, D), lambda i: (i, 0)),
        out_shape=jax.ShapeDtypeStruct((NP, D), jnp.float32),
    )(x_pad, ws[0], dinv)

    for i in range(NUM_LAYERS):
        p = _agg_sc(s, r3, c3)
        if i < NUM_LAYERS - 1:
            s = pl.pallas_call(
                _layer_body,
                grid=grid,
                in_specs=[pl.BlockSpec((2, RB, D), lambda i: (0, i, 0)),
                          pl.BlockSpec((RB, D), lambda i: (i, 0)),
                          pl.BlockSpec((RB, 1), lambda i: (i, 0)),
                          pl.BlockSpec((1, D), lambda i: (0, 0)),
                          pl.BlockSpec((D, D), lambda i: (0, 0))],
                out_specs=pl.BlockSpec((RB, D), lambda i: (i, 0)),
                out_shape=jax.ShapeDtypeStruct((NP, D), jnp.float32),
            )(p, s, dinv, bs[i].reshape(1, D), ws[i + 1])
        else:
            sums, cnt = pl.pallas_call(
                _pool_body,
                grid=grid,
                in_specs=[pl.BlockSpec((2, RB, D), lambda i: (0, i, 0)),
                          pl.BlockSpec((RB, D), lambda i: (i, 0)),
                          pl.BlockSpec((RB, 1), lambda i: (i, 0)),
                          pl.BlockSpec((1, D), lambda i: (0, 0)),
                          pl.BlockSpec((RB, 1), lambda i: (i, 0))],
                out_specs=[pl.BlockSpec((G, D), lambda i: (0, 0)),
                           pl.BlockSpec((G, 1), lambda i: (0, 0))],
                out_shape=[jax.ShapeDtypeStruct((G, D), jnp.float32),
                           jax.ShapeDtypeStruct((G, 1), jnp.float32)],
            )(p, s, dinv, bs[i].reshape(1, D), batch2)

    out = pl.pallas_call(
        _lin_body,
        out_shape=jax.ShapeDtypeStruct((G, G), jnp.float32),
    )(sums, cnt, lin_w, lin_b.reshape(1, G))
    return out


# speedup vs baseline: 4.0240x; 1.3682x over previous
import functools

import jax
import jax.numpy as jnp
from jax import lax
from jax.experimental import pallas as pl
from jax.experimental.pallas import tpu as pltpu
from jax.experimental.pallas import tpu_sc as plsc

N = 10000
E = 320000
D = 128
G = 64
NUM_LAYERS = 6
NP = 10240
NW = 32
NCH = 80
CH = 128
EP = NW * NCH * CH
RPT = NP // 16
RB = 2048
DEGW = 16

_mesh = plsc.VectorSubcoreMesh(core_axis_name="c", subcore_axis_name="s")


NRC = RPT // CH


def _fill_rowidx(rowidx, sid):
    base = sid * RPT
    for j in range(NRC):
        for m in range(CH // 16):
            rowidx[j, pl.ds(m * 16, 16)] = (
                base + j * CH + m * 16 + lax.iota(jnp.int32, 16))


@functools.partial(
    pl.kernel,
    out_type=jax.ShapeDtypeStruct((2, NP, DEGW), jnp.float32),
    mesh=_mesh,
    scratch_types=[
        pltpu.VMEM((CH,), jnp.int32),
        pltpu.VMEM((NRC, CH), jnp.int32),
        pltpu.VMEM((CH, DEGW), jnp.float32),
        pltpu.VMEM((CH, DEGW), jnp.float32),
        pltpu.VMEM_SHARED((NP, DEGW), jnp.float32),
    ])
def _deg_sc(c_hbm, out_hbm, cidx, rowidx, ones_v, zrows, acc):
    cid = lax.axis_index("c")
    sid = lax.axis_index("s")
    wid = sid * 2 + cid

    def fill_bufs(i, carry):
        ones_v[i, pl.ds(0, DEGW)] = jnp.ones((DEGW,), jnp.float32)
        zrows[i, pl.ds(0, DEGW)] = jnp.zeros((DEGW,), jnp.float32)
        return carry
    lax.fori_loop(0, CH, fill_bufs, 0)
    _fill_rowidx(rowidx, sid)

    for j in range(NRC):
        pltpu.sync_copy(zrows, acc.at[rowidx.at[j]])
    plsc.subcore_barrier()

    def body(k, carry):
        pltpu.sync_copy(c_hbm.at[wid, k], cidx)
        pltpu.sync_copy(ones_v, acc.at[cidx], add=True)
        return carry
    lax.fori_loop(0, NCH, body, 0)

    plsc.subcore_barrier()
    for j in range(NRC):
        pltpu.sync_copy(acc.at[rowidx.at[j]], zrows)
        pltpu.sync_copy(zrows, out_hbm.at[cid, pl.ds(sid * RPT + j * CH, CH)])


@functools.partial(
    pl.kernel,
    out_type=jax.ShapeDtypeStruct((2, NP, D), jnp.float32),
    mesh=_mesh,
    scratch_types=[
        pltpu.VMEM((2, CH), jnp.int32),
        pltpu.VMEM((2, CH), jnp.int32),
        pltpu.VMEM((NRC, CH), jnp.int32),
        pltpu.VMEM((CH, D), jnp.float32),
        pltpu.VMEM((CH, D), jnp.float32),
        pltpu.VMEM_SHARED((NP, D), jnp.float32),
        pltpu.SemaphoreType.DMA,
        pltpu.SemaphoreType.DMA,
    ])
def _agg_sc(s_hbm, e_hbm, out_hbm, ibuf0, ibuf1, rowidx, rows0, rows1,
            acc, sg0, sg1):
    cid = lax.axis_index("c")
    sid = lax.axis_index("s")
    wid = sid * 2 + cid

    def fill_zero(i, carry):
        for j in range(D // 16):
            rows0[i, pl.ds(j * 16, 16)] = jnp.zeros((16,), jnp.float32)
        return carry
    lax.fori_loop(0, CH, fill_zero, 0)
    _fill_rowidx(rowidx, sid)

    for j in range(NRC):
        pltpu.sync_copy(rows0, acc.at[rowidx.at[j]])
    plsc.subcore_barrier()

    pltpu.sync_copy(e_hbm.at[wid, 0], ibuf0)
    pltpu.async_copy(s_hbm.at[ibuf0.at[0]], rows0, sg0)

    def body(h, carry):
        k1 = 2 * h + 1
        pltpu.make_async_copy(s_hbm.at[ibuf0.at[0]], rows0, sg0).wait()
        pltpu.sync_copy(e_hbm.at[wid, k1], ibuf1)
        pltpu.async_copy(s_hbm.at[ibuf1.at[0]], rows1, sg1)
        pltpu.sync_copy(rows0, acc.at[ibuf0.at[1]], add=True)
        pltpu.sync_copy(e_hbm.at[wid, k1 + 1], ibuf0)
        pltpu.make_async_copy(s_hbm.at[ibuf1.at[0]], rows1, sg1).wait()
        pltpu.async_copy(s_hbm.at[ibuf0.at[0]], rows0, sg0)
        pltpu.sync_copy(rows1, acc.at[ibuf1.at[1]], add=True)
        return carry
    lax.fori_loop(0, NCH // 2, body, 0)
    pltpu.make_async_copy(s_hbm.at[ibuf0.at[0]], rows0, sg0).wait()

    plsc.subcore_barrier()
    for j in range(NRC):
        pltpu.sync_copy(acc.at[rowidx.at[j]], rows0)
        pltpu.sync_copy(rows0, out_hbm.at[cid, pl.ds(sid * RPT + j * CH, CH)])


def _prep_body(degp_ref, dinv_ref):
    d = degp_ref[0, :, 0:1] + degp_ref[1, :, 0:1] + 1.0
    dinv_ref[...] = lax.rsqrt(d)


def _mm0_body(x_ref, w_ref, dinv_ref, o_ref):
    o_ref[...] = dinv_ref[...] * jnp.dot(
        x_ref[...], w_ref[...], preferred_element_type=jnp.float32)


def _layer_body(p_ref, s_ref, dinv_ref, b_ref, w_ref, o_ref):
    di = dinv_ref[...]
    h = jnp.maximum(di * (p_ref[0] + p_ref[1] + s_ref[...]) + b_ref[...], 0.0)
    o_ref[...] = di * jnp.dot(h, w_ref[...], preferred_element_type=jnp.float32)


def _pool_body(p_ref, s_ref, dinv_ref, b_ref, batch_ref, sums_ref, cnt_ref):
    @pl.when(pl.program_id(0) == 0)
    def _():
        sums_ref[...] = jnp.zeros_like(sums_ref)
        cnt_ref[...] = jnp.zeros_like(cnt_ref)
    di = dinv_ref[...]
    h = jnp.maximum(di * (p_ref[0] + p_ref[1] + s_ref[...]) + b_ref[...], 0.0)
    oh = (batch_ref[...] == lax.broadcasted_iota(jnp.int32, (1, G), 1)
          ).astype(jnp.float32)
    sums_ref[...] += lax.dot_general(
        oh, h, (((0,), (0,)), ((), ())), preferred_element_type=jnp.float32)
    cnt_ref[...] += lax.dot_general(
        oh, jnp.ones((RB, 1), jnp.float32), (((0,), (0,)), ((), ())),
        preferred_element_type=jnp.float32)


def _lin_body(sums_ref, cnt_ref, w_ref, b_ref, o_ref):
    pooled = sums_ref[...] / jnp.maximum(cnt_ref[...], 1.0)
    o_ref[...] = jnp.dot(
        pooled, w_ref[...], preferred_element_type=jnp.float32) + b_ref[...]


def kernel(x, edge_index, batch, w0, w1, w2, w3, w4, w5,
           b0, b1, b2, b3, b4, b5, lin_w, lin_b):
    ws = [w0, w1, w2, w3, w4, w5]
    bs = [b0, b1, b2, b3, b4, b5]
    r = edge_index[0]
    c = edge_index[1]
    idx_pad = N + (jnp.arange(EP - E, dtype=jnp.int32) % (NP - N))
    r3 = jnp.concatenate([r, idx_pad]).reshape(NW, NCH, CH)
    c3 = jnp.concatenate([c, idx_pad]).reshape(NW, NCH, CH)
    e3 = jnp.stack([r3, c3], axis=2)
    extra = (N + (jnp.arange(NW * 2 * CH, dtype=jnp.int32) % (NP - N))
             ).reshape(NW, 1, 2, CH)
    e3 = jnp.concatenate([e3, extra], axis=1)
    x_pad = jnp.pad(x, ((0, NP - N), (0, 0)))
    batch2 = jnp.pad(batch, (0, NP - N), constant_values=G).reshape(NP, 1)

    degp = _deg_sc(c3)

    dinv = pl.pallas_call(
        _prep_body,
        out_shape=jax.ShapeDtypeStruct((NP, 1), jnp.float32),
    )(degp)

    grid = (NP // RB,)
    s = pl.pallas_call(
        _mm0_body,
        grid=grid,
        in_specs=[pl.BlockSpec((RB, D), lambda i: (i, 0)),
                  pl.BlockSpec((D, D), lambda i: (0, 0)),
                  pl.BlockSpec((RB, 1), lambda i: (i, 0))],
        out_specs=pl.BlockSpec((RB, D), lambda i: (i, 0)),
        out_shape=jax.ShapeDtypeStruct((NP, D), jnp.float32),
    )(x_pad, ws[0], dinv)

    for i in range(NUM_LAYERS):
        p = _agg_sc(s, e3)
        if i < NUM_LAYERS - 1:
            s = pl.pallas_call(
                _layer_body,
                grid=grid,
                in_specs=[pl.BlockSpec((2, RB, D), lambda i: (0, i, 0)),
                          pl.BlockSpec((RB, D), lambda i: (i, 0)),
                          pl.BlockSpec((RB, 1), lambda i: (i, 0)),
                          pl.BlockSpec((1, D), lambda i: (0, 0)),
                          pl.BlockSpec((D, D), lambda i: (0, 0))],
                out_specs=pl.BlockSpec((RB, D), lambda i: (i, 0)),
                out_shape=jax.ShapeDtypeStruct((NP, D), jnp.float32),
            )(p, s, dinv, bs[i].reshape(1, D), ws[i + 1])
        else:
            sums, cnt = pl.pallas_call(
                _pool_body,
                grid=grid,
                in_specs=[pl.BlockSpec((2, RB, D), lambda i: (0, i, 0)),
                          pl.BlockSpec((RB, D), lambda i: (i, 0)),
                          pl.BlockSpec((RB, 1), lambda i: (i, 0)),
                          pl.BlockSpec((1, D), lambda i: (0, 0)),
                          pl.BlockSpec((RB, 1), lambda i: (i, 0))],
                out_specs=[pl.BlockSpec((G, D), lambda i: (0, 0)),
                           pl.BlockSpec((G, 1), lambda i: (0, 0))],
                out_shape=[jax.ShapeDtypeStruct((G, D), jnp.float32),
                           jax.ShapeDtypeStruct((G, 1), jnp.float32)],
            )(p, s, dinv, bs[i].reshape(1, D), batch2)

    out = pl.pallas_call(
        _lin_body,
        out_shape=jax.ShapeDtypeStruct((G, G), jnp.float32),
    )(sums, cnt, lin_w, lin_b.reshape(1, G))
    return out


# speedup vs baseline: 4.0279x; 1.0010x over previous
import functools

import jax
import jax.numpy as jnp
from jax import lax
from jax.experimental import pallas as pl
from jax.experimental.pallas import tpu as pltpu
from jax.experimental.pallas import tpu_sc as plsc

N = 10000
E = 320000
D = 128
G = 64
NUM_LAYERS = 6
NP = 10240
NW = 32
NCH = 80
CH = 128
EP = NW * NCH * CH
RPT = NP // 16
RB = 2048
DEGW = 16

_mesh = plsc.VectorSubcoreMesh(core_axis_name="c", subcore_axis_name="s")


NRC = RPT // CH


def _fill_rowidx(rowidx, sid):
    base = sid * RPT
    for j in range(NRC):
        for m in range(CH // 16):
            rowidx[j, pl.ds(m * 16, 16)] = (
                base + j * CH + m * 16 + lax.iota(jnp.int32, 16))


@functools.partial(
    pl.kernel,
    out_type=jax.ShapeDtypeStruct((2, NP, DEGW), jnp.float32),
    mesh=_mesh,
    scratch_types=[
        pltpu.VMEM((CH,), jnp.int32),
        pltpu.VMEM((NRC, CH), jnp.int32),
        pltpu.VMEM((CH, DEGW), jnp.float32),
        pltpu.VMEM((CH, DEGW), jnp.float32),
        pltpu.VMEM_SHARED((NP, DEGW), jnp.float32),
    ])
def _deg_sc(c_hbm, out_hbm, cidx, rowidx, ones_v, zrows, acc):
    cid = lax.axis_index("c")
    sid = lax.axis_index("s")
    wid = sid * 2 + cid

    def fill_bufs(i, carry):
        ones_v[i, pl.ds(0, DEGW)] = jnp.ones((DEGW,), jnp.float32)
        zrows[i, pl.ds(0, DEGW)] = jnp.zeros((DEGW,), jnp.float32)
        return carry
    lax.fori_loop(0, CH, fill_bufs, 0)
    _fill_rowidx(rowidx, sid)

    for j in range(NRC):
        pltpu.sync_copy(zrows, acc.at[rowidx.at[j]])
    plsc.subcore_barrier()

    def body(k, carry):
        pltpu.sync_copy(c_hbm.at[wid, k], cidx)
        pltpu.sync_copy(ones_v, acc.at[cidx], add=True)
        return carry
    lax.fori_loop(0, NCH, body, 0)

    plsc.subcore_barrier()
    for j in range(NRC):
        pltpu.sync_copy(acc.at[rowidx.at[j]], zrows)
        pltpu.sync_copy(zrows, out_hbm.at[cid, pl.ds(sid * RPT + j * CH, CH)])


@functools.partial(
    pl.kernel,
    out_type=jax.ShapeDtypeStruct((2, NP, D), jnp.float32),
    mesh=_mesh,
    scratch_types=[
        pltpu.VMEM((2, CH), jnp.int32),
        pltpu.VMEM((2, CH), jnp.int32),
        pltpu.VMEM((NRC, CH), jnp.int32),
        pltpu.VMEM((CH, D), jnp.float32),
        pltpu.VMEM((CH, D), jnp.float32),
        pltpu.VMEM_SHARED((NP, D), jnp.float32),
        pltpu.SemaphoreType.DMA,
        pltpu.SemaphoreType.DMA,
    ])
def _agg_sc(s_hbm, e_hbm, out_hbm, ibuf0, ibuf1, rowidx, rows0, rows1,
            acc, sg0, sg1):
    cid = lax.axis_index("c")
    sid = lax.axis_index("s")
    wid = sid * 2 + cid

    def fill_zero(i, carry):
        for j in range(D // 16):
            rows0[i, pl.ds(j * 16, 16)] = jnp.zeros((16,), jnp.float32)
        return carry
    lax.fori_loop(0, CH, fill_zero, 0)
    _fill_rowidx(rowidx, sid)

    for j in range(NRC):
        pltpu.sync_copy(rows0, acc.at[rowidx.at[j]])
    plsc.subcore_barrier()

    pltpu.sync_copy(e_hbm.at[wid, 0], ibuf0)
    pltpu.async_copy(s_hbm.at[ibuf0.at[0]], rows0, sg0)

    def body(h, carry):
        k1 = 2 * h + 1
        pltpu.make_async_copy(s_hbm.at[ibuf0.at[0]], rows0, sg0).wait()
        pltpu.sync_copy(e_hbm.at[wid, k1], ibuf1)
        pltpu.async_copy(s_hbm.at[ibuf1.at[0]], rows1, sg1)
        pltpu.sync_copy(rows0, acc.at[ibuf0.at[1]], add=True)
        pltpu.sync_copy(e_hbm.at[wid, k1 + 1], ibuf0)
        pltpu.make_async_copy(s_hbm.at[ibuf1.at[0]], rows1, sg1).wait()
        pltpu.async_copy(s_hbm.at[ibuf0.at[0]], rows0, sg0)
        pltpu.sync_copy(rows1, acc.at[ibuf1.at[1]], add=True)
        return carry
    lax.fori_loop(0, NCH // 2, body, 0)
    pltpu.make_async_copy(s_hbm.at[ibuf0.at[0]], rows0, sg0).wait()

    plsc.subcore_barrier()
    for j in range(NRC):
        pltpu.sync_copy(acc.at[rowidx.at[j]], rows0)
        pltpu.sync_copy(rows0, out_hbm.at[cid, pl.ds(sid * RPT + j * CH, CH)])


def _prep_body(degp_ref, dinv_ref):
    d = degp_ref[0, :, 0:1] + degp_ref[1, :, 0:1] + 1.0
    dinv_ref[...] = lax.rsqrt(d)


def _mm0_body(x_ref, w_ref, dinv_ref, o_ref):
    o_ref[...] = dinv_ref[...] * jnp.dot(
        x_ref[...], w_ref[...], preferred_element_type=jnp.float32)


def _layer_body(p_ref, s_ref, dinv_ref, b_ref, w_ref, o_ref):
    di = dinv_ref[...]
    h = jnp.maximum(di * (p_ref[0] + p_ref[1] + s_ref[...]) + b_ref[...], 0.0)
    o_ref[...] = di * jnp.dot(h, w_ref[...], preferred_element_type=jnp.float32)


def _pool_body(p_ref, s_ref, dinv_ref, b_ref, batch_ref, sums_ref, cnt_ref):
    @pl.when(pl.program_id(0) == 0)
    def _():
        sums_ref[...] = jnp.zeros_like(sums_ref)
        cnt_ref[...] = jnp.zeros_like(cnt_ref)
    di = dinv_ref[...]
    h = jnp.maximum(di * (p_ref[0] + p_ref[1] + s_ref[...]) + b_ref[...], 0.0)
    oh = (batch_ref[...] == lax.broadcasted_iota(jnp.int32, (1, G), 1)
          ).astype(jnp.float32)
    sums_ref[...] += lax.dot_general(
        oh, h, (((0,), (0,)), ((), ())), preferred_element_type=jnp.float32)
    cnt_ref[...] += lax.dot_general(
        oh, jnp.ones((RB, 1), jnp.float32), (((0,), (0,)), ((), ())),
        preferred_element_type=jnp.float32)


def _lin_body(sums_ref, cnt_ref, w_ref, b_ref, o_ref):
    pooled = sums_ref[...] / jnp.maximum(cnt_ref[...], 1.0)
    o_ref[...] = jnp.dot(
        pooled, w_ref[...], preferred_element_type=jnp.float32) + b_ref[...]


def kernel(x, edge_index, batch, w0, w1, w2, w3, w4, w5,
           b0, b1, b2, b3, b4, b5, lin_w, lin_b):
    ws = [w0, w1, w2, w3, w4, w5]
    bs = [b0, b1, b2, b3, b4, b5]
    r = edge_index[0]
    c = edge_index[1]
    idx_pad = N + (jnp.arange(EP - E, dtype=jnp.int32) % (NP - N))
    r3 = jnp.concatenate([r, idx_pad]).reshape(NW, NCH, CH)
    c3 = jnp.concatenate([c, idx_pad]).reshape(NW, NCH, CH)
    e3 = jnp.stack([r3, c3], axis=2)
    extra = (N + (jnp.arange(NW * 2 * 2 * CH, dtype=jnp.int32) % (NP - N))
             ).reshape(NW, 2, 2, CH)
    e3 = jnp.concatenate([e3, extra], axis=1)
    x_pad = jnp.pad(x, ((0, NP - N), (0, 0)))
    batch2 = jnp.pad(batch, (0, NP - N), constant_values=G).reshape(NP, 1)

    degp = _deg_sc(c3)

    dinv = pl.pallas_call(
        _prep_body,
        out_shape=jax.ShapeDtypeStruct((NP, 1), jnp.float32),
    )(degp)

    grid = (NP // RB,)
    s = pl.pallas_call(
        _mm0_body,
        grid=grid,
        in_specs=[pl.BlockSpec((RB, D), lambda i: (i, 0)),
                  pl.BlockSpec((D, D), lambda i: (0, 0)),
                  pl.BlockSpec((RB, 1), lambda i: (i, 0))],
        out_specs=pl.BlockSpec((RB, D), lambda i: (i, 0)),
        out_shape=jax.ShapeDtypeStruct((NP, D), jnp.float32),
    )(x_pad, ws[0], dinv)

    for i in range(NUM_LAYERS):
        p = _agg_sc(s, e3)
        if i < NUM_LAYERS - 1:
            s = pl.pallas_call(
                _layer_body,
                grid=grid,
                in_specs=[pl.BlockSpec((2, RB, D), lambda i: (0, i, 0)),
                          pl.BlockSpec((RB, D), lambda i: (i, 0)),
                          pl.BlockSpec((RB, 1), lambda i: (i, 0)),
                          pl.BlockSpec((1, D), lambda i: (0, 0)),
                          pl.BlockSpec((D, D), lambda i: (0, 0))],
                out_specs=pl.BlockSpec((RB, D), lambda i: (i, 0)),
                out_shape=jax.ShapeDtypeStruct((NP, D), jnp.float32),
            )(p, s, dinv, bs[i].reshape(1, D), ws[i + 1])
        else:
            sums, cnt = pl.pallas_call(
                _pool_body,
                grid=grid,
                in_specs=[pl.BlockSpec((2, RB, D), lambda i: (0, i, 0)),
                          pl.BlockSpec((RB, D), lambda i: (i, 0)),
                          pl.BlockSpec((RB, 1), lambda i: (i, 0)),
                          pl.BlockSpec((1, D), lambda i: (0, 0)),
                          pl.BlockSpec((RB, 1), lambda i: (i, 0))],
                out_specs=[pl.BlockSpec((G, D), lambda i: (0, 0)),
                           pl.BlockSpec((G, 1), lambda i: (0, 0))],
                out_shape=[jax.ShapeDtypeStruct((G, D), jnp.float32),
                           jax.ShapeDtypeStruct((G, 1), jnp.float32)],
            )(p, s, dinv, bs[i].reshape(1, D), batch2)

    out = pl.pallas_call(
        _lin_body,
        out_shape=jax.ShapeDtypeStruct((G, G), jnp.float32),
    )(sums, cnt, lin_w, lin_b.reshape(1, G))
    return out
